# baseline (speedup 1.0000x reference)
import functools

import jax
import jax.numpy as jnp
import numpy as np
from jax import lax
from jax.experimental import pallas as pl
from jax.experimental.pallas import tpu as pltpu
from jax.experimental.pallas import tpu_sc as plsc

_B, _T, _D = 128, 2048, 128
_MAXT = 4096
_NC, _NS = 2, 16
_NW = _NC * _NS
_BPW = _B // _NW
_CHUNK = 128
_GRP = 256
_GPB = _T // _GRP
_CPG = _GRP // _CHUNK
_IPB = _T // _CHUNK
_ZCH = 128

def _threefry2x32(k0, k1, c0, c1):
    rot = ((13, 15, 26, 6), (17, 29, 16, 24))
    ks = (np.uint32(k0), np.uint32(k1),
          np.uint32(k0) ^ np.uint32(k1) ^ np.uint32(0x1BD11BDA))
    x0 = (c0 + ks[0]).astype(np.uint32)
    x1 = (c1 + ks[1]).astype(np.uint32)
    for i in range(5):
        for r in rot[i % 2]:
            x0 = (x0 + x1).astype(np.uint32)
            x1 = ((x1 << np.uint32(r)) | (x1 >> np.uint32(32 - r))).astype(np.uint32)
            x1 = x1 ^ x0
        x0 = (x0 + ks[(i + 1) % 3]).astype(np.uint32)
        x1 = (x1 + ks[(i + 2) % 3] + np.uint32(i + 1)).astype(np.uint32)
    return x0, x1


def _np_split(kp, num):
    b0, b1 = _threefry2x32(kp[0], kp[1], np.zeros(num, np.uint32),
                           np.arange(num, dtype=np.uint32))
    return np.stack([b0, b1], axis=1)


def _np_bits32(kp, n):
    b0, b1 = _threefry2x32(kp[0], kp[1], np.zeros(n, np.uint32),
                           np.arange(n, dtype=np.uint32))
    return b0 ^ b1


def _np_permutation(kp, n):
    x = np.arange(n, dtype=np.int32)
    rounds = int(np.ceil(3 * np.log(n) / np.log(np.iinfo(np.uint32).max)))
    for _ in range(rounds):
        ks = _np_split(kp, 2)
        kp, sub = ks[0], ks[1]
        x = x[np.argsort(_np_bits32(sub, n), kind="stable")]
    return x


def _global_row_indices() -> np.ndarray:
    keys = _np_split(np.array([0, 0], np.uint32), _B)
    gidx = np.stack([_np_permutation(keys[b], _T) for b in range(_B)])
    gidx += (np.arange(_B, dtype=np.int32) * _T)[:, None]
    return gidx.reshape(_B * _IPB, _CHUNK)


_GIDX = _global_row_indices()


def _shuffle_body(x_hbm, gidx_hbm, out_hbm, idx_v, gbuf, sem_g, sem_o):
    wid = lax.axis_index("s") * _NC + lax.axis_index("c")
    b0 = wid * _BPW

    pltpu.sync_copy(gidx_hbm.at[pl.ds(b0 * _IPB, _BPW * _IPB)], idx_v)

    out_pending = [None, None]
    for bb in range(_BPW):
        b = b0 + bb
        for g in range(_GPB):
            pbuf = g % 2
            if out_pending[pbuf] is not None:
                out_pending[pbuf].wait()
            gh = []
            for q in range(_CPG):
                gh.append(pltpu.async_copy(
                    x_hbm.at[idx_v.at[bb * _IPB + g * _CPG + q]],
                    gbuf.at[pbuf, pl.ds(q * _CHUNK, _CHUNK)],
                    sem_g))
            for h in gh:
                h.wait()
            out_pending[pbuf] = pltpu.async_copy(
                gbuf.at[pbuf],
                out_hbm.at[pl.ds(b * _MAXT + g * _GRP, _GRP)],
                sem_o)
    for h in out_pending:
        if h is not None:
            h.wait()


def _pad_body(x_ref, o_ref):
    o_ref[...] = jnp.zeros((_T, _D), jnp.float32)


_kernel_cache = None


def _shuffle():
    global _kernel_cache
    if _kernel_cache is None:
        _kernel_cache = pl.kernel(
            _shuffle_body,
            out_type=jax.ShapeDtypeStruct((_B * _MAXT, _D), jnp.float32),
            mesh=plsc.VectorSubcoreMesh(
                core_axis_name="c", subcore_axis_name="s",
                num_cores=_NC, num_subcores=_NS),
            scratch_types=[
                pltpu.VMEM((_BPW * _IPB, _CHUNK), jnp.int32),
                pltpu.VMEM((2, _GRP, _D), jnp.float32),
                pltpu.SemaphoreType.DMA,
                pltpu.SemaphoreType.DMA,
            ],
        )
    return _kernel_cache


_pad_zeros = pl.pallas_call(
    _pad_body,
    out_shape=jax.ShapeDtypeStruct((_B * _MAXT, _D), jnp.float32),
    grid=(_B,),
    in_specs=[pl.BlockSpec(memory_space=pl.ANY)],
    out_specs=pl.BlockSpec((_T, _D), lambda i: (2 * i + 1, 0)),
    input_output_aliases={0: 0},
)


def kernel(x):
    shuffled = _shuffle()(x.reshape(_B * _T, _D), jnp.asarray(_GIDX))
    return _pad_zeros(shuffled).reshape(_B, _MAXT, _D)


# speedup vs baseline: 1.1449x; 1.1449x over previous
import functools

import jax
import jax.numpy as jnp
import numpy as np
from jax import lax
from jax.experimental import pallas as pl
from jax.experimental.pallas import tpu as pltpu
from jax.experimental.pallas import tpu_sc as plsc

_B, _T, _D = 128, 2048, 128
_MAXT = 4096
_NC, _NS = 2, 16
_NW = _NC * _NS
_BPW = _B // _NW
_CHUNK = 128
_GRP = 256
_GPB = _T // _GRP
_CPG = _GRP // _CHUNK
_IPB = _T // _CHUNK
_ZCH = 128

def _threefry2x32(k0, k1, c0, c1):
    rot = ((13, 15, 26, 6), (17, 29, 16, 24))
    ks = (np.uint32(k0), np.uint32(k1),
          np.uint32(k0) ^ np.uint32(k1) ^ np.uint32(0x1BD11BDA))
    x0 = (c0 + ks[0]).astype(np.uint32)
    x1 = (c1 + ks[1]).astype(np.uint32)
    for i in range(5):
        for r in rot[i % 2]:
            x0 = (x0 + x1).astype(np.uint32)
            x1 = ((x1 << np.uint32(r)) | (x1 >> np.uint32(32 - r))).astype(np.uint32)
            x1 = x1 ^ x0
        x0 = (x0 + ks[(i + 1) % 3]).astype(np.uint32)
        x1 = (x1 + ks[(i + 2) % 3] + np.uint32(i + 1)).astype(np.uint32)
    return x0, x1


def _np_split(kp, num):
    b0, b1 = _threefry2x32(kp[0], kp[1], np.zeros(num, np.uint32),
                           np.arange(num, dtype=np.uint32))
    return np.stack([b0, b1], axis=1)


def _np_bits32(kp, n):
    b0, b1 = _threefry2x32(kp[0], kp[1], np.zeros(n, np.uint32),
                           np.arange(n, dtype=np.uint32))
    return b0 ^ b1


def _np_permutation(kp, n):
    x = np.arange(n, dtype=np.int32)
    rounds = int(np.ceil(3 * np.log(n) / np.log(np.iinfo(np.uint32).max)))
    for _ in range(rounds):
        ks = _np_split(kp, 2)
        kp, sub = ks[0], ks[1]
        x = x[np.argsort(_np_bits32(sub, n), kind="stable")]
    return x


def _global_row_indices() -> np.ndarray:
    keys = _np_split(np.array([0, 0], np.uint32), _B)
    gidx = np.stack([_np_permutation(keys[b], _T) for b in range(_B)])
    gidx += (np.arange(_B, dtype=np.int32) * _T)[:, None]
    return gidx.reshape(_B * _IPB, _CHUNK)


_GIDX = _global_row_indices()


def _shuffle_pad_body(x_hbm, gidx_hbm, out_hbm, idx_v, zbuf, gbuf, zshared,
                      sem_g, sem_o, sem_z):
    cid = lax.axis_index("c")
    sid = lax.axis_index("s")
    wid = sid * _NC + cid
    b0 = wid * _BPW

    z = jnp.zeros((16,), jnp.float32)

    def zero_row(i, carry):
        for j in range(_D // 16):
            zbuf[i, pl.ds(j * 16, 16)] = z
        return carry

    lax.fori_loop(0, _ZCH, zero_row, 0)
    pltpu.sync_copy(zbuf, zshared.at[pl.ds(sid * _ZCH, _ZCH)])
    plsc.subcore_barrier()

    zero_pending = []
    for bb in range(_BPW):
        b = b0 + bb
        zero_pending.append(pltpu.async_copy(
            zshared, out_hbm.at[pl.ds(b * _MAXT + _T, _T)], sem_z))

    pltpu.sync_copy(gidx_hbm.at[pl.ds(b0 * _IPB, _BPW * _IPB)], idx_v)

    out_pending = [None, None]
    for bb in range(_BPW):
        b = b0 + bb
        for g in range(_GPB):
            pbuf = g % 2
            if out_pending[pbuf] is not None:
                out_pending[pbuf].wait()
            gh = []
            for q in range(_CPG):
                gh.append(pltpu.async_copy(
                    x_hbm.at[idx_v.at[bb * _IPB + g * _CPG + q]],
                    gbuf.at[pbuf, pl.ds(q * _CHUNK, _CHUNK)],
                    sem_g))
            for h in gh:
                h.wait()
            out_pending[pbuf] = pltpu.async_copy(
                gbuf.at[pbuf],
                out_hbm.at[pl.ds(b * _MAXT + g * _GRP, _GRP)],
                sem_o)
    for h in out_pending:
        if h is not None:
            h.wait()
    for h in zero_pending:
        h.wait()


_kernel_cache = None


def _shuffle():
    global _kernel_cache
    if _kernel_cache is None:
        _kernel_cache = pl.kernel(
            _shuffle_pad_body,
            out_type=jax.ShapeDtypeStruct((_B * _MAXT, _D), jnp.float32),
            mesh=plsc.VectorSubcoreMesh(
                core_axis_name="c", subcore_axis_name="s",
                num_cores=_NC, num_subcores=_NS),
            scratch_types=[
                pltpu.VMEM((_BPW * _IPB, _CHUNK), jnp.int32),
                pltpu.VMEM((_ZCH, _D), jnp.float32),
                pltpu.VMEM((2, _GRP, _D), jnp.float32),
                pltpu.VMEM_SHARED((_NS * _ZCH, _D), jnp.float32),
                pltpu.SemaphoreType.DMA,
                pltpu.SemaphoreType.DMA,
                pltpu.SemaphoreType.DMA,
            ],
        )
    return _kernel_cache


def kernel(x):
    out = _shuffle()(x.reshape(_B * _T, _D), jnp.asarray(_GIDX))
    return out.reshape(_B, _MAXT, _D)


# speedup vs baseline: 1.1543x; 1.0082x over previous
import functools

import jax
import jax.numpy as jnp
import numpy as np
from jax import lax
from jax.experimental import pallas as pl
from jax.experimental.pallas import tpu as pltpu
from jax.experimental.pallas import tpu_sc as plsc

_B, _T, _D = 128, 2048, 128
_MAXT = 4096
_NC, _NS = 2, 16
_NW = _NC * _NS
_BPW = _B // _NW
_CHUNK = 128
_GRP = 256
_GPB = _T // _GRP
_CPG = _GRP // _CHUNK
_IPB = _T // _CHUNK
_ZCH = 128

def _threefry2x32(k0, k1, c0, c1):
    rot = ((13, 15, 26, 6), (17, 29, 16, 24))
    ks = (np.uint32(k0), np.uint32(k1),
          np.uint32(k0) ^ np.uint32(k1) ^ np.uint32(0x1BD11BDA))
    x0 = (c0 + ks[0]).astype(np.uint32)
    x1 = (c1 + ks[1]).astype(np.uint32)
    for i in range(5):
        for r in rot[i % 2]:
            x0 = (x0 + x1).astype(np.uint32)
            x1 = ((x1 << np.uint32(r)) | (x1 >> np.uint32(32 - r))).astype(np.uint32)
            x1 = x1 ^ x0
        x0 = (x0 + ks[(i + 1) % 3]).astype(np.uint32)
        x1 = (x1 + ks[(i + 2) % 3] + np.uint32(i + 1)).astype(np.uint32)
    return x0, x1


def _np_split(kp, num):
    b0, b1 = _threefry2x32(kp[0], kp[1], np.zeros(num, np.uint32),
                           np.arange(num, dtype=np.uint32))
    return np.stack([b0, b1], axis=1)


def _np_bits32(kp, n):
    b0, b1 = _threefry2x32(kp[0], kp[1], np.zeros(n, np.uint32),
                           np.arange(n, dtype=np.uint32))
    return b0 ^ b1


def _np_permutation(kp, n):
    x = np.arange(n, dtype=np.int32)
    rounds = int(np.ceil(3 * np.log(n) / np.log(np.iinfo(np.uint32).max)))
    for _ in range(rounds):
        ks = _np_split(kp, 2)
        kp, sub = ks[0], ks[1]
        x = x[np.argsort(_np_bits32(sub, n), kind="stable")]
    return x


def _global_row_indices() -> np.ndarray:
    keys = _np_split(np.array([0, 0], np.uint32), _B)
    gidx = np.stack([_np_permutation(keys[b], _T) for b in range(_B)])
    gidx += (np.arange(_B, dtype=np.int32) * _T)[:, None]
    return gidx.reshape(_B * _IPB, _CHUNK)


_GIDX = _global_row_indices()


def _shuffle_pad_body(x_hbm, gidx_hbm, out_hbm, idx_v, zbuf, gbuf, zshared,
                      sem_g, sem_o, sem_z):
    cid = lax.axis_index("c")
    sid = lax.axis_index("s")
    wid = sid * _NC + cid
    b0 = wid * _BPW

    z = jnp.zeros((16,), jnp.float32)

    def zero_row(i, carry):
        for j in range(_D // 16):
            zbuf[i, pl.ds(j * 16, 16)] = z
        return carry

    pltpu.sync_copy(gidx_hbm.at[pl.ds(b0 * _IPB, _BPW * _IPB)], idx_v)

    groups = [(b0 + bb, g) for bb in range(_BPW) for g in range(_GPB)]
    out_pending = [None, None]
    prev = None

    def fire(i):
        pbuf = i % 2
        if out_pending[pbuf] is not None:
            out_pending[pbuf].wait()
            out_pending[pbuf] = None
        gh = []
        for q in range(_CPG):
            gh.append(pltpu.async_copy(
                x_hbm.at[idx_v.at[(groups[i][0] - b0) * _IPB
                                  + groups[i][1] * _CPG + q]],
                gbuf.at[pbuf, pl.ds(q * _CHUNK, _CHUNK)],
                sem_g))
        return gh

    def drain(i, gh):
        pbuf = i % 2
        for h in gh:
            h.wait()
        b, g = groups[i]
        out_pending[pbuf] = pltpu.async_copy(
            gbuf.at[pbuf],
            out_hbm.at[pl.ds(b * _MAXT + g * _GRP, _GRP)],
            sem_o)

    prev = fire(0)

    lax.fori_loop(0, _ZCH, zero_row, 0)
    pltpu.sync_copy(zbuf, zshared.at[pl.ds(sid * _ZCH, _ZCH)])
    plsc.subcore_barrier()
    zero_pending = []
    for bb in range(_BPW):
        b = b0 + bb
        zero_pending.append(pltpu.async_copy(
            zshared, out_hbm.at[pl.ds(b * _MAXT + _T, _T)], sem_z))

    for i in range(1, len(groups)):
        gh = fire(i)
        drain(i - 1, prev)
        prev = gh
    drain(len(groups) - 1, prev)
    for h in out_pending:
        if h is not None:
            h.wait()
    for h in zero_pending:
        h.wait()


_kernel_cache = None


def _shuffle():
    global _kernel_cache
    if _kernel_cache is None:
        _kernel_cache = pl.kernel(
            _shuffle_pad_body,
            out_type=jax.ShapeDtypeStruct((_B * _MAXT, _D), jnp.float32),
            mesh=plsc.VectorSubcoreMesh(
                core_axis_name="c", subcore_axis_name="s",
                num_cores=_NC, num_subcores=_NS),
            scratch_types=[
                pltpu.VMEM((_BPW * _IPB, _CHUNK), jnp.int32),
                pltpu.VMEM((_ZCH, _D), jnp.float32),
                pltpu.VMEM((2, _GRP, _D), jnp.float32),
                pltpu.VMEM_SHARED((_NS * _ZCH, _D), jnp.float32),
                pltpu.SemaphoreType.DMA,
                pltpu.SemaphoreType.DMA,
                pltpu.SemaphoreType.DMA,
            ],
        )
    return _kernel_cache


def kernel(x):
    out = _shuffle()(x.reshape(_B * _T, _D), jnp.asarray(_GIDX))
    return out.reshape(_B, _MAXT, _D)
